# baseline (speedup 1.0000x reference)
import jax
import jax.numpy as jnp
from jax import lax
from jax.experimental import pallas as pl
from jax.experimental.pallas import tpu as pltpu
from jax.experimental.pallas import tpu_sc as plsc

_NC, _NS = 2, 16
_NW = _NC * _NS
_R = 8
_L = 16
_TC_B = 3
_BS = 2048


def _sc_body(x_hbm, pe_hbm, out_hbm, pebuf, xbuf, lsem, ssem):
    pe_elems = pe_hbm.shape[0]
    x_off = x_hbm.shape[0] - pe_elems
    chunk = _R * 1024
    seq_elems_per_w = pe_elems // _NW
    n_chunks = seq_elems_per_w // chunk

    wid = lax.axis_index("s") * _NC + lax.axis_index("c")
    base = wid * seq_elems_per_w

    def start_loads(c, pb):
        e0 = base + c * chunk
        pltpu.make_async_copy(
            pe_hbm.at[pl.ds(e0, chunk)], pebuf.at[pb], lsem
        ).start()
        pltpu.make_async_copy(
            x_hbm.at[pl.ds(x_off + e0, chunk)], xbuf.at[pb], lsem
        ).start()

    def wait_loads(pb):
        pltpu.make_async_copy(pe_hbm.at[pl.ds(0, chunk)], pebuf.at[pb], lsem).wait()
        pltpu.make_async_copy(x_hbm.at[pl.ds(0, chunk)], xbuf.at[pb], lsem).wait()

    def start_store(c, pb):
        e0 = base + c * chunk
        pltpu.make_async_copy(
            xbuf.at[pb], out_hbm.at[pl.ds(e0, chunk)], ssem
        ).start()

    def drain_one_store(pb):
        pltpu.make_async_copy(
            xbuf.at[pb], out_hbm.at[pl.ds(0, chunk)], ssem
        ).wait()

    start_loads(0, 0)

    def step(c, _):
        pb = lax.rem(c, 2)
        wait_loads(pb)

        @pl.when(c + 1 < n_chunks)
        def _():
            @pl.when(c >= 1)
            def _():
                drain_one_store(1 - pb)

            start_loads(c + 1, 1 - pb)

        @plsc.parallel_loop(0, chunk, _L, unroll=8)
        def _(v):
            xbuf[pb, pl.ds(v, _L)] = xbuf[pb, pl.ds(v, _L)] + pebuf[pb, pl.ds(v, _L)]

        start_store(c, pb)
        return 0

    lax.fori_loop(0, n_chunks, step, 0)

    for _i in range(2):
        drain_one_store(0)


def _tc_body(x_ref, pe_ref, out_ref):
    out_ref[0] = x_ref[0] + pe_ref[...]


def kernel(x, pe_table):
    B, S, H = x.shape
    sc_b = B - _TC_B
    x1d = x.reshape(B * S * H)
    pe1d = pe_table[:S].reshape(S * H)

    mesh = plsc.VectorSubcoreMesh(
        core_axis_name="c", subcore_axis_name="s", num_cores=_NC, num_subcores=_NS
    )
    sc_out = pl.kernel(
        _sc_body,
        out_type=jax.ShapeDtypeStruct((sc_b * S * H,), x.dtype),
        mesh=mesh,
        scratch_types=[
            pltpu.VMEM((2, _R * 1024), x.dtype),
            pltpu.VMEM((2, _R * 1024), x.dtype),
            pltpu.SemaphoreType.DMA,
            pltpu.SemaphoreType.DMA,
        ],
    )(x1d, pe1d)

    tc_out = pl.pallas_call(
        _tc_body,
        grid=(S // _BS, _TC_B),
        in_specs=[
            pl.BlockSpec((1, _BS, H), lambda s, b: (b, s, 0)),
            pl.BlockSpec((_BS, H), lambda s, b: (s, 0)),
        ],
        out_specs=pl.BlockSpec((1, _BS, H), lambda s, b: (b, s, 0)),
        out_shape=jax.ShapeDtypeStruct((_TC_B, S, H), x.dtype),
    )(x, pe_table)

    return jnp.concatenate([tc_out, sc_out.reshape(sc_b, S, H)], axis=0)


# speedup vs baseline: 1.5643x; 1.5643x over previous
import jax
import jax.numpy as jnp
from jax import lax
from jax.experimental import pallas as pl
from jax.experimental.pallas import tpu as pltpu
from jax.experimental.pallas import tpu_sc as plsc

_NC, _NS = 2, 16
_NW = _NC * _NS
_R = 16
_L = 16
_TC_B = 3
_BS = 2048


def _sc_body(x_hbm, pe_hbm, out_hbm, pebuf, xbuf, lsem, ssem):
    pe_rows = pe_hbm.shape[0]
    h = pe_hbm.shape[1]
    row0 = x_hbm.shape[0] - pe_rows
    seq_per_w = pe_rows // _NW
    n_chunks = seq_per_w // _R

    wid = lax.axis_index("s") * _NC + lax.axis_index("c")
    seq0 = wid * seq_per_w

    def start_loads(c, pb):
        s0 = seq0 + c * _R
        pltpu.make_async_copy(
            pe_hbm.at[pl.ds(s0, _R), :], pebuf.at[pb], lsem
        ).start()
        pltpu.make_async_copy(
            x_hbm.at[pl.ds(row0 + s0, _R), :], xbuf.at[pb], lsem
        ).start()

    def wait_loads(pb):
        pltpu.make_async_copy(pe_hbm.at[pl.ds(0, _R), :], pebuf.at[pb], lsem).wait()
        pltpu.make_async_copy(x_hbm.at[pl.ds(0, _R), :], xbuf.at[pb], lsem).wait()

    def start_store(c, pb):
        s0 = seq0 + c * _R
        pltpu.make_async_copy(
            xbuf.at[pb], out_hbm.at[pl.ds(s0, _R), :], ssem
        ).start()

    def drain_one_store(pb):
        pltpu.make_async_copy(
            xbuf.at[pb], out_hbm.at[pl.ds(0, _R), :], ssem
        ).wait()

    start_loads(0, 0)

    def step(c, _):
        pb = lax.rem(c, 2)
        wait_loads(pb)

        @pl.when(c + 1 < n_chunks)
        def _():
            @pl.when(c >= 1)
            def _():
                drain_one_store(1 - pb)

            start_loads(c + 1, 1 - pb)

        for r in range(_R):
            @plsc.parallel_loop(0, h, _L, unroll=8)
            def _(j, r=r):
                xbuf[pb, r, pl.ds(j, _L)] = (
                    xbuf[pb, r, pl.ds(j, _L)] + pebuf[pb, r, pl.ds(j, _L)]
                )

        start_store(c, pb)
        return 0

    lax.fori_loop(0, n_chunks, step, 0)

    for _i in range(2):
        drain_one_store(0)


def _tc_body(x_ref, pe_ref, out_ref):
    out_ref[0] = x_ref[0] + pe_ref[...]


def kernel(x, pe_table):
    B, S, H = x.shape
    sc_b = B - _TC_B
    x2d = x.reshape(B * S, H)

    mesh = plsc.VectorSubcoreMesh(
        core_axis_name="c", subcore_axis_name="s", num_cores=_NC, num_subcores=_NS
    )
    sc_out = pl.kernel(
        _sc_body,
        out_type=jax.ShapeDtypeStruct((sc_b * S, H), x.dtype),
        mesh=mesh,
        scratch_types=[
            pltpu.VMEM((2, _R, H), x.dtype),
            pltpu.VMEM((2, _R, H), x.dtype),
            pltpu.SemaphoreType.DMA,
            pltpu.SemaphoreType.DMA,
        ],
    )(x2d, pe_table)

    tc_out = pl.pallas_call(
        _tc_body,
        grid=(S // _BS, _TC_B),
        in_specs=[
            pl.BlockSpec((1, _BS, H), lambda s, b: (b, s, 0)),
            pl.BlockSpec((_BS, H), lambda s, b: (s, 0)),
        ],
        out_specs=pl.BlockSpec((1, _BS, H), lambda s, b: (b, s, 0)),
        out_shape=jax.ShapeDtypeStruct((_TC_B, S, H), x.dtype),
    )(x, pe_table)

    return jnp.concatenate([tc_out, sc_out.reshape(sc_b, S, H)], axis=0)
